# baseline (speedup 1.0000x reference)
import dataclasses

import jax
import jax.numpy as jnp
from jax.experimental import pallas as pl
from jax.experimental.pallas import tpu as pltpu
from jax.experimental.pallas import tpu_sc as plsc

_B, _L = 1024, 32
_V, _D, _LV = 8192, 64, 1000
_BL = _B * _L

_DP = 128
_TOK_WIN = 256
_CTX_WIN = 128

_ROWS = 512
_NBATCH = _ROWS // _L

_SC_MESH = plsc.VectorSubcoreMesh(core_axis_name="c", subcore_axis_name="s")

_SC_PARAMS = pltpu.CompilerParams()
if "needs_layout_passes" in pltpu.CompilerParams.__dataclass_fields__:
    _SC_PARAMS = dataclasses.replace(_SC_PARAMS, needs_layout_passes=False)


def _gather_pipeline(table_hbm, idx_hbm, out_hbm, n_rows, win):
    def body(i_vmem, o_vmem):
        pltpu.sync_copy(table_hbm.at[i_vmem.at[0]], o_vmem)

    pltpu.emit_pipeline(
        body,
        grid=(n_rows // win,),
        in_specs=[pl.BlockSpec((1, win), index_map=lambda i: (0, i))],
        out_specs=[pl.BlockSpec((win, _DP), index_map=lambda i: (i, 0))],
        core_axis_name=("c", "s"),
        dimension_semantics=(pltpu.PARALLEL,),
    )(idx_hbm, out_hbm)


def _tok_pipeline(tok_hbm, ids_hbm, out_hbm, scratch):
    def body(i_vmem, o_vmem):
        pltpu.sync_copy(tok_hbm.at[i_vmem.at[0]], scratch)

        @pl.loop(0, _TOK_WIN // 2)
        def _(q):
            @pl.loop(0, 2)
            def _(h):
                @pl.loop(0, 2)
                def _(c):
                    a = scratch[2 * q + h, pl.ds(c * 32, 16)]
                    b = scratch[2 * q + h, pl.ds(c * 32 + 16, 16)]
                    v = plsc.pack(a, b, format=plsc.PackFormat.INTERLEAVED)
                    o_vmem[q, pl.ds(h * 64 + c * 32, 32)] = v

    pltpu.emit_pipeline(
        body,
        grid=(_BL // _TOK_WIN,),
        in_specs=[pl.BlockSpec((1, _TOK_WIN), index_map=lambda i: (0, i))],
        out_specs=[pl.BlockSpec((_TOK_WIN // 2, _DP), index_map=lambda i: (i, 0))],
        core_axis_name=("c", "s"),
        dimension_semantics=(pltpu.PARALLEL,),
    )(ids_hbm, out_hbm)


def _sc_gather_body(tok_hbm, ids_hbm, lab_hbm, ctx_hbm, otok_hbm, olab_hbm,
                    scratch):
    _tok_pipeline(tok_hbm, ids_hbm, otok_hbm, scratch)
    _gather_pipeline(lab_hbm, ctx_hbm, olab_hbm, _B, _CTX_WIN)


def _sc_gather(tok_pad, ids_flat, lab_pad, ctx_flat):
    f = pl.kernel(
        _sc_gather_body,
        out_type=(
            jax.ShapeDtypeStruct((_BL // 2, _DP), jnp.bfloat16),
            jax.ShapeDtypeStruct((_B, _DP), jnp.float32),
        ),
        mesh=_SC_MESH,
        scratch_types=[pltpu.VMEM((_TOK_WIN, _DP), jnp.float32)],
        compiler_params=_SC_PARAMS,
    )
    return f(tok_pad, ids_flat, lab_pad, ctx_flat)


def _proj_body(tok_ref, lab_ref, wt_ref, b_ref, o_ref):
    v = tok_ref[...]
    y = jnp.stack([v[:, :_D], v[:, _D:]], axis=1).reshape(_ROWS, _D)
    tok = y.astype(jnp.float32).reshape(_NBATCH, _L, _D)
    lab = lab_ref[...][:, :_D]
    x = (tok + lab[:, None, :]).reshape(_ROWS, _D).astype(jnp.bfloat16)
    acc = jnp.dot(x, wt_ref[...], preferred_element_type=jnp.float32)
    o_ref[...] = acc + b_ref[...]


def _project(tok_x, lab_x, wt, b2d):
    return pl.pallas_call(
        _proj_body,
        grid=(_BL // _ROWS,),
        in_specs=[
            pl.BlockSpec((_ROWS // 2, _DP), lambda i: (i, 0)),
            pl.BlockSpec((_NBATCH, _DP), lambda i: (i, 0)),
            pl.BlockSpec((_D, _V), lambda i: (0, 0)),
            pl.BlockSpec((1, _V), lambda i: (0, 0)),
        ],
        out_specs=pl.BlockSpec((_ROWS, _V), lambda i: (i, 0)),
        out_shape=jax.ShapeDtypeStruct((_BL, _V), jnp.float32),
        compiler_params=pltpu.CompilerParams(
            dimension_semantics=("arbitrary",),
        ),
    )(tok_x, lab_x, wt, b2d)


def kernel(input_ids, context, token_embed, label_embed, W, b):
    wt = W.astype(jnp.bfloat16).T
    ids_flat = input_ids.reshape(1, _BL).astype(jnp.int32)
    ctx_flat = context.reshape(1, _B).astype(jnp.int32)
    tok_pad = jnp.pad(token_embed, ((0, 0), (0, _DP - _D)))
    lab_pad = jnp.pad(label_embed, ((0, 0), (0, _DP - _D)))
    tok_x, lab_x = _sc_gather(tok_pad, ids_flat, lab_pad, ctx_flat)
    logits = _project(tok_x, lab_x, wt, b.reshape(1, _V))
    return logits.reshape(_B, _L, _V)


# speedup vs baseline: 1.0093x; 1.0093x over previous
import jax
import jax.numpy as jnp
from jax.experimental import pallas as pl
from jax.experimental.pallas import tpu as pltpu
from jax.experimental.pallas import tpu_sc as plsc

_B, _L = 1024, 32
_V, _D, _LV = 8192, 64, 1000
_BL = _B * _L

_DP = 128
_TOK_WIN = 256
_CTX_WIN = 128

_ROWS = 512
_NBATCH = _ROWS // _L

_SC_MESH = plsc.VectorSubcoreMesh(core_axis_name="c", subcore_axis_name="s")


def _gather_pipeline(table_hbm, idx_hbm, out_hbm, n_rows, win):
    def body(i_vmem, o_vmem):
        pltpu.sync_copy(table_hbm.at[i_vmem.at[0]], o_vmem)

    pltpu.emit_pipeline(
        body,
        grid=(n_rows // win,),
        in_specs=[pl.BlockSpec((1, win), index_map=lambda i: (0, i))],
        out_specs=[pl.BlockSpec((win, _DP), index_map=lambda i: (i, 0))],
        core_axis_name=("c", "s"),
        dimension_semantics=(pltpu.PARALLEL,),
    )(idx_hbm, out_hbm)


def _sc_gather_body(tok_hbm, ids_hbm, lab_hbm, ctx_hbm, otok_hbm, olab_hbm):
    _gather_pipeline(tok_hbm, ids_hbm, otok_hbm, _BL, _TOK_WIN)
    _gather_pipeline(lab_hbm, ctx_hbm, olab_hbm, _B, _CTX_WIN)


def _sc_gather(tok_pad, ids_flat, lab_pad, ctx_flat):
    f = pl.kernel(
        _sc_gather_body,
        out_type=(
            jax.ShapeDtypeStruct((_BL, _DP), jnp.float32),
            jax.ShapeDtypeStruct((_B, _DP), jnp.float32),
        ),
        mesh=_SC_MESH,
    )
    return f(tok_pad, ids_flat, lab_pad, ctx_flat)


def _proj_body(tok_ref, lab_ref, wt_ref, b_ref, o_ref):
    tok = tok_ref[...][:, :_D].reshape(_NBATCH, _L, _D)
    lab = lab_ref[...][:, :_D]
    x = (tok + lab[:, None, :]).reshape(_ROWS, _D).astype(jnp.bfloat16)
    acc = jnp.dot(x, wt_ref[...], preferred_element_type=jnp.float32)
    o_ref[...] = acc + b_ref[...]


def _project(tok_x, lab_x, wt, b2d):
    return pl.pallas_call(
        _proj_body,
        grid=(_BL // _ROWS,),
        in_specs=[
            pl.BlockSpec((_ROWS, _DP), lambda i: (i, 0)),
            pl.BlockSpec((_NBATCH, _DP), lambda i: (i, 0)),
            pl.BlockSpec((_D, _V), lambda i: (0, 0)),
            pl.BlockSpec((1, _V), lambda i: (0, 0)),
        ],
        out_specs=pl.BlockSpec((_ROWS, _V), lambda i: (i, 0)),
        out_shape=jax.ShapeDtypeStruct((_BL, _V), jnp.float32),
        compiler_params=pltpu.CompilerParams(
            dimension_semantics=("arbitrary",),
        ),
    )(tok_x, lab_x, wt, b2d)


def kernel(input_ids, context, token_embed, label_embed, W, b):
    wt = W.astype(jnp.bfloat16).T
    ids_flat = input_ids.reshape(1, _BL).astype(jnp.int32)
    ctx_flat = context.reshape(1, _B).astype(jnp.int32)
    tok_pad = jnp.pad(token_embed, ((0, 0), (0, _DP - _D)))
    lab_pad = jnp.pad(label_embed, ((0, 0), (0, _DP - _D)))
    tok_x, lab_x = _sc_gather(tok_pad, ids_flat, lab_pad, ctx_flat)
    logits = _project(tok_x, lab_x, wt, b.reshape(1, _V))
    return logits.reshape(_B, _L, _V)
